# baseline (speedup 1.0000x reference)
import functools

import jax
import jax.numpy as jnp
from jax import lax
from jax.experimental import pallas as pl
from jax.experimental.pallas import tpu as pltpu
from jax.experimental.pallas import tpu_sc as plsc

_NC = 2
_NS = 16
_NW = _NC * _NS
_IW = 128
_RING = 4


@functools.lru_cache(maxsize=None)
def _make_lookup(n_tokens: int, vocab: int, d_model: int):
    assert d_model % 16 == 0
    assert n_tokens % (_NW * _IW * _RING) == 0
    per_w = n_tokens // _NW
    n_halves = per_w // _IW
    n_steps = n_halves // _RING
    tok_rows = per_w // _IW

    mesh = plsc.VectorSubcoreMesh(core_axis_name="c", subcore_axis_name="s")

    @functools.partial(
        pl.kernel,
        mesh=mesh,
        out_type=jax.ShapeDtypeStruct((n_tokens, d_model), jnp.float32),
        scratch_types=[
            pltpu.VMEM((vocab, d_model), jnp.float32),
            pltpu.VMEM((tok_rows, _IW), jnp.int32),
            pltpu.VMEM((_RING, _IW, d_model), jnp.float32),
        ] + [pltpu.SemaphoreType.DMA] * _RING,
        compiler_params=pltpu.CompilerParams(needs_layout_passes=False),
    )
    def lookup(tok_hbm, tab_hbm, out_hbm, tab_v, tok_v, rows_v, *sem_o):
        wid = lax.axis_index("s") * _NC + lax.axis_index("c")
        row_base = wid * per_w
        pltpu.sync_copy(tab_hbm, tab_v)
        pltpu.sync_copy(tok_hbm.at[pl.ds(wid * tok_rows, tok_rows)], tok_v)

        def out_copy(h, sl):
            return pltpu.make_async_copy(
                rows_v.at[sl],
                out_hbm.at[pl.ds(row_base + h * _IW, _IW)],
                sem_o[sl],
            )

        def fill(h, sl):
            def grp_body(grp, carry):
                toks = tok_v[h, pl.ds(grp * 16, 16)]
                base = grp * 16
                for i in range(16):
                    s = toks[i]
                    for c in range(d_model // 16):
                        rows_v[sl, base + i, pl.ds(c * 16, 16)] = (
                            tab_v[s, pl.ds(c * 16, 16)])
                return carry

            lax.fori_loop(0, _IW // 16, grp_body, 0, unroll=False)

        def run_step(h0, first):
            for sl in range(_RING):
                if not first:
                    out_copy(h0 + sl - _RING, sl).wait()
                fill(h0 + sl, sl)
                out_copy(h0 + sl, sl).start()

        run_step(0, True)
        lax.fori_loop(
            1, n_steps,
            lambda s, c: (run_step(s * _RING, False), c)[1], 0,
            unroll=False)
        for sl in range(_RING):
            out_copy(n_halves - _RING + sl, sl).wait()

    return lookup


def kernel(tokens, embed_weight):
    b, k, l = tokens.shape
    vocab, d_model = embed_weight.shape
    n = b * k * l
    tok_2d = tokens.reshape((n // _IW, _IW)).astype(jnp.int32)
    out = _make_lookup(n, vocab, d_model)(tok_2d, embed_weight)
    return out.reshape((b, k, l, d_model))


# speedup vs baseline: 4.0315x; 4.0315x over previous
import functools

import jax
import jax.numpy as jnp
from jax import lax
from jax.experimental import pallas as pl
from jax.experimental.pallas import tpu as pltpu
from jax.experimental.pallas import tpu_sc as plsc

_NC = 2
_NS = 16
_NW = _NC * _NS
_CHUNK = 256
_IW = 64


_RING = 8


@functools.lru_cache(maxsize=None)
def _make_lookup(n_tokens: int, vocab: int, d_model: int):
    assert n_tokens % (_NW * _IW * _RING) == 0
    per_w = n_tokens // _NW
    n_halves = per_w // _IW
    n_steps = n_halves // _RING
    tok_rows = per_w // _IW

    mesh = plsc.VectorSubcoreMesh(core_axis_name="c", subcore_axis_name="s")

    @functools.partial(
        pl.kernel,
        mesh=mesh,
        out_type=jax.ShapeDtypeStruct((n_tokens, d_model), jnp.float32),
        scratch_types=[
            pltpu.VMEM_SHARED((vocab, d_model), jnp.float32),
            pltpu.VMEM((tok_rows, _IW), jnp.int32),
            pltpu.VMEM((_RING, _IW, d_model), jnp.float32),
            pltpu.SemaphoreType.DMA,
        ] + [pltpu.SemaphoreType.DMA] * _RING,
        compiler_params=pltpu.CompilerParams(needs_layout_passes=False),
    )
    def lookup(tok_hbm, tab_hbm, out_hbm, tab_v, tok_v, rows_v, sem_g, *sem_o):
        wid = lax.axis_index("s") * _NC + lax.axis_index("c")
        row_base = wid * per_w
        @pl.when(lax.axis_index("s") == 0)
        def _():
            pltpu.sync_copy(tab_hbm, tab_v)

        pltpu.sync_copy(tok_hbm.at[pl.ds(wid * tok_rows, tok_rows)], tok_v)
        plsc.subcore_barrier()

        def gather_start(h, sl):
            return pltpu.make_async_copy(
                tab_v.at[tok_v.at[h]], rows_v.at[sl], sem_g)

        def out_copy(h, sl):
            return pltpu.make_async_copy(
                rows_v.at[sl],
                out_hbm.at[pl.ds(row_base + h * _IW, _IW)],
                sem_o[sl],
            )

        def run_step(h0, first):
            gathers = []
            for sl in range(_RING):
                if not first:
                    out_copy(h0 + sl - _RING, sl).wait()
                cp = gather_start(h0 + sl, sl)
                cp.start()
                gathers.append(cp)
            for sl in range(_RING):
                gathers[sl].wait()
                out_copy(h0 + sl, sl).start()

        run_step(0, True)
        lax.fori_loop(
            1, n_steps,
            lambda s, c: (run_step(s * _RING, False), c)[1], 0,
            unroll=False)
        for sl in range(_RING):
            out_copy(n_halves - _RING + sl, sl).wait()

    return lookup


def kernel(tokens, embed_weight):
    b, k, l = tokens.shape
    vocab, d_model = embed_weight.shape
    n = b * k * l
    tok_2d = tokens.reshape((n // _IW, _IW)).astype(jnp.int32)
    out = _make_lookup(n, vocab, d_model)(tok_2d, embed_weight)
    return out.reshape((b, k, l, d_model))


# speedup vs baseline: 4.1319x; 1.0249x over previous
import functools

import jax
import jax.numpy as jnp
from jax import lax
from jax.experimental import pallas as pl
from jax.experimental.pallas import tpu as pltpu
from jax.experimental.pallas import tpu_sc as plsc

_NC = 2
_NS = 16
_NW = _NC * _NS
_CHUNK = 256
_IW = 128


_RING = 4


@functools.lru_cache(maxsize=None)
def _make_lookup(n_tokens: int, vocab: int, d_model: int):
    assert n_tokens % (_NW * _IW * _RING) == 0
    per_w = n_tokens // _NW
    n_halves = per_w // _IW
    n_steps = n_halves // _RING
    tok_rows = per_w // _IW

    mesh = plsc.VectorSubcoreMesh(core_axis_name="c", subcore_axis_name="s")

    @functools.partial(
        pl.kernel,
        mesh=mesh,
        out_type=jax.ShapeDtypeStruct((n_tokens, d_model), jnp.float32),
        scratch_types=[
            pltpu.VMEM_SHARED((vocab, d_model), jnp.float32),
            pltpu.VMEM((tok_rows, _IW), jnp.int32),
            pltpu.VMEM((_RING, _IW, d_model), jnp.float32),
            pltpu.SemaphoreType.DMA,
        ] + [pltpu.SemaphoreType.DMA] * _RING,
        compiler_params=pltpu.CompilerParams(needs_layout_passes=False),
    )
    def lookup(tok_hbm, tab_hbm, out_hbm, tab_v, tok_v, rows_v, sem_g, *sem_o):
        wid = lax.axis_index("s") * _NC + lax.axis_index("c")
        row_base = wid * per_w
        @pl.when(lax.axis_index("s") == 0)
        def _():
            pltpu.sync_copy(tab_hbm, tab_v)

        pltpu.sync_copy(tok_hbm.at[pl.ds(wid * tok_rows, tok_rows)], tok_v)
        plsc.subcore_barrier()

        def gather_start(h, sl):
            return pltpu.make_async_copy(
                tab_v.at[tok_v.at[h]], rows_v.at[sl], sem_g)

        def out_copy(h, sl):
            return pltpu.make_async_copy(
                rows_v.at[sl],
                out_hbm.at[pl.ds(row_base + h * _IW, _IW)],
                sem_o[sl],
            )

        def run_step(h0, first):
            gathers = []
            for sl in range(_RING):
                if not first:
                    out_copy(h0 + sl - _RING, sl).wait()
                cp = gather_start(h0 + sl, sl)
                cp.start()
                gathers.append(cp)
            for sl in range(_RING):
                gathers[sl].wait()
                out_copy(h0 + sl, sl).start()

        run_step(0, True)
        lax.fori_loop(
            1, n_steps,
            lambda s, c: (run_step(s * _RING, False), c)[1], 0,
            unroll=False)
        for sl in range(_RING):
            out_copy(n_halves - _RING + sl, sl).wait()

    return lookup


def kernel(tokens, embed_weight):
    b, k, l = tokens.shape
    vocab, d_model = embed_weight.shape
    n = b * k * l
    tok_2d = tokens.reshape((n // _IW, _IW)).astype(jnp.int32)
    out = _make_lookup(n, vocab, d_model)(tok_2d, embed_weight)
    return out.reshape((b, k, l, d_model))
